# baseline (speedup 1.0000x reference)
import functools

import jax
import jax.numpy as jnp
from jax import lax
from jax.experimental import pallas as pl
from jax.experimental.pallas import tpu as pltpu
from jax.experimental.pallas import tpu_sc as plsc

_B = 16384
_D = 4096
_NC = 2
_NS = 16
_NW = _NC * _NS
_RPW = _B // _NW
_C = 8
_W = 1024
_NH = _D // _W
_NCH = (_RPW // _C) * _NH
_NBUF = 4
_LANES = 16


def _body(x_hbm, idx_hbm, m_hbm, out_hbm,
          idx_v, x_v, g_v, m_v, gsem, lsem, osem):
    wid = lax.axis_index("s") * _NC + lax.axis_index("c")
    base = wid * _RPW
    pltpu.sync_copy(idx_hbm.at[pl.ds(base, _RPW)], idx_v)

    m32 = m_hbm.bitcast(jnp.int32)

    def chunk_coords(ci):
        row0 = base + (ci // _NH) * _C
        col0 = pl.multiple_of((ci % _NH) * _W, _W)
        q0 = pl.multiple_of(row0 >> 2, 2)
        return row0, col0, q0

    def issue_in(ci, b):
        row0, col0, q0 = chunk_coords(ci)
        pltpu.async_copy(
            x_hbm.at[idx_v.at[pl.ds((ci // _NH) * _C, _C)],
                     pl.ds(col0, _W)],
            g_v.at[b], gsem.at[b])
        pltpu.async_copy(x_hbm.at[pl.ds(row0, _C), pl.ds(col0, _W)],
                         x_v.at[b], lsem.at[b])
        pltpu.async_copy(m32.at[pl.ds(q0, _C // 4), pl.ds(col0, _W)],
                         m_v.at[b], lsem.at[b])

    def wait_in(ci, b):
        row0, col0, q0 = chunk_coords(ci)
        pltpu.make_async_copy(
            x_hbm.at[idx_v.at[pl.ds((ci // _NH) * _C, _C)],
                     pl.ds(col0, _W)],
            g_v.at[b], gsem.at[b]).wait()
        pltpu.make_async_copy(x_hbm.at[pl.ds(row0, _C), pl.ds(col0, _W)],
                              x_v.at[b], lsem.at[b]).wait()
        pltpu.make_async_copy(m32.at[pl.ds(q0, _C // 4), pl.ds(col0, _W)],
                              m_v.at[b], lsem.at[b]).wait()

    def compute(b):
        def group(e, _):
            off = e * _LANES
            for q in range(_C // 4):
                mw = m_v[b, q, pl.ds(off, _LANES)]
                for s in range(4):
                    r = 4 * q + s
                    keep = (mw & jnp.int32(1 << (8 * s))) != 0
                    xv = x_v[b, r, pl.ds(off, _LANES)]
                    gv = g_v[b, r, pl.ds(off, _LANES)]
                    g_v[b, r, pl.ds(off, _LANES)] = jnp.where(keep, gv, xv)
            return ()

        lax.fori_loop(0, _W // _LANES, group, (), unroll=False)

    def issue_out(ci, b):
        row0, col0, _q = chunk_coords(ci)
        pltpu.async_copy(g_v.at[b],
                         out_hbm.at[pl.ds(row0, _C), pl.ds(col0, _W)],
                         osem.at[b])

    def wait_out(ci, b):
        row0, col0, _q = chunk_coords(ci)
        pltpu.make_async_copy(g_v.at[b],
                              out_hbm.at[pl.ds(row0, _C), pl.ds(col0, _W)],
                              osem.at[b]).wait()

    issue_in(0, 0)
    issue_in(1, 1)

    def step(i, _):
        for b in range(_NBUF):
            ci = _NBUF * i + b

            @pl.when(ci >= 2)
            def _():
                wait_out(ci - 2, (ci - 2) % _NBUF)

            @pl.when(ci + 2 < _NCH)
            def _():
                issue_in(ci + 2, (ci + 2) % _NBUF)

            wait_in(ci, b)
            compute(b)
            issue_out(ci, b)
        return ()

    lax.fori_loop(0, _NCH // _NBUF, step, (), unroll=False)
    wait_out(_NCH - 2, (_NCH - 2) % _NBUF)
    wait_out(_NCH - 1, (_NCH - 1) % _NBUF)


@jax.jit
def _cutmix_sc(x, idx, m8):
    mesh = plsc.VectorSubcoreMesh(core_axis_name="c", subcore_axis_name="s",
                                  num_cores=_NC, num_subcores=_NS)
    run = pl.kernel(
        _body,
        out_type=jax.ShapeDtypeStruct((_B, _D), jnp.float32),
        mesh=mesh,
        scratch_types=[
            pltpu.VMEM((_RPW,), jnp.int32),
            pltpu.VMEM((_NBUF, _C, _W), jnp.float32),
            pltpu.VMEM((_NBUF, _C, _W), jnp.float32),
            pltpu.VMEM((_NBUF, _C // 4, _W), jnp.int32),
            pltpu.SemaphoreType.DMA((_NBUF,)),
            pltpu.SemaphoreType.DMA((_NBUF,)),
            pltpu.SemaphoreType.DMA((_NBUF,)),
        ],
    )
    return run(x, idx, m8)


def kernel(x, shuffled_idx, mask):
    return _cutmix_sc(x, shuffled_idx, mask.view(jnp.int8))


# speedup vs baseline: 1.0444x; 1.0444x over previous
import functools

import jax
import jax.numpy as jnp
from jax import lax
from jax.experimental import pallas as pl
from jax.experimental.pallas import tpu as pltpu
from jax.experimental.pallas import tpu_sc as plsc

_B = 16384
_D = 4096
_NC = 2
_NS = 16
_NW = _NC * _NS
_RPW = _B // _NW
_C = 8
_W = 2048
_NH = _D // _W
_NCH = (_RPW // _C) * _NH
_NBUF = 3
_LANES = 16


def _body(x_hbm, idx_hbm, m_hbm, out_hbm,
          idx_v, x_v, g_v, m_v, gsem, lsem, osem):
    wid = lax.axis_index("s") * _NC + lax.axis_index("c")
    base = wid * _RPW
    pltpu.sync_copy(idx_hbm.at[pl.ds(base, _RPW)], idx_v)

    m32 = m_hbm.bitcast(jnp.int32)

    def chunk_coords(ci):
        row0 = base + (ci // _NH) * _C
        col0 = pl.multiple_of((ci % _NH) * _W, _W)
        q0 = pl.multiple_of(row0 >> 2, 2)
        return row0, col0, q0

    def issue_in(ci, b):
        row0, col0, q0 = chunk_coords(ci)
        pltpu.async_copy(
            x_hbm.at[idx_v.at[pl.ds((ci // _NH) * _C, _C)],
                     pl.ds(col0, _W)],
            g_v.at[b], gsem.at[b])
        pltpu.async_copy(x_hbm.at[pl.ds(row0, _C), pl.ds(col0, _W)],
                         x_v.at[b], lsem.at[b])
        pltpu.async_copy(m32.at[pl.ds(q0, _C // 4), pl.ds(col0, _W)],
                         m_v.at[b], lsem.at[b])

    def wait_in(ci, b):
        row0, col0, q0 = chunk_coords(ci)
        pltpu.make_async_copy(
            x_hbm.at[idx_v.at[pl.ds((ci // _NH) * _C, _C)],
                     pl.ds(col0, _W)],
            g_v.at[b], gsem.at[b]).wait()
        pltpu.make_async_copy(x_hbm.at[pl.ds(row0, _C), pl.ds(col0, _W)],
                              x_v.at[b], lsem.at[b]).wait()
        pltpu.make_async_copy(m32.at[pl.ds(q0, _C // 4), pl.ds(col0, _W)],
                              m_v.at[b], lsem.at[b]).wait()

    def compute(b):
        def group(e, _):
            off = e * _LANES
            for q in range(_C // 4):
                mw = m_v[b, q, pl.ds(off, _LANES)]
                for s in range(4):
                    r = 4 * q + s
                    keep = (mw & jnp.int32(1 << (8 * s))) != 0
                    xv = x_v[b, r, pl.ds(off, _LANES)]
                    gv = g_v[b, r, pl.ds(off, _LANES)]
                    g_v[b, r, pl.ds(off, _LANES)] = jnp.where(keep, gv, xv)
            return ()

        lax.fori_loop(0, _W // _LANES, group, (), unroll=False)

    def issue_out(ci, b):
        row0, col0, _q = chunk_coords(ci)
        pltpu.async_copy(g_v.at[b],
                         out_hbm.at[pl.ds(row0, _C), pl.ds(col0, _W)],
                         osem.at[b])

    def wait_out(ci, b):
        row0, col0, _q = chunk_coords(ci)
        pltpu.make_async_copy(g_v.at[b],
                              out_hbm.at[pl.ds(row0, _C), pl.ds(col0, _W)],
                              osem.at[b]).wait()

    issue_in(0, 0)
    issue_in(1, 1)

    _NMAIN = (_NCH // _NBUF) * _NBUF

    def step(i, _):
        for b in range(_NBUF):
            ci = _NBUF * i + b

            @pl.when(ci >= 1)
            def _():
                wait_out(ci - 1, (ci - 1) % _NBUF)

            @pl.when(ci + 2 < _NCH)
            def _():
                issue_in(ci + 2, (ci + 2) % _NBUF)

            wait_in(ci, b)
            compute(b)
            issue_out(ci, b)
        return ()

    lax.fori_loop(0, _NMAIN // _NBUF, step, (), unroll=False)
    for ci in range(_NMAIN, _NCH):
        b = ci % _NBUF
        wait_out(ci - 1, (ci - 1) % _NBUF)
        if ci + 2 < _NCH:
            issue_in(ci + 2, (ci + 2) % _NBUF)
        wait_in(ci, b)
        compute(b)
        issue_out(ci, b)
    wait_out(_NCH - 1, (_NCH - 1) % _NBUF)


@jax.jit
def _cutmix_sc(x, idx, m8):
    mesh = plsc.VectorSubcoreMesh(core_axis_name="c", subcore_axis_name="s",
                                  num_cores=_NC, num_subcores=_NS)
    run = pl.kernel(
        _body,
        out_type=jax.ShapeDtypeStruct((_B, _D), jnp.float32),
        mesh=mesh,
        scratch_types=[
            pltpu.VMEM((_RPW,), jnp.int32),
            pltpu.VMEM((_NBUF, _C, _W), jnp.float32),
            pltpu.VMEM((_NBUF, _C, _W), jnp.float32),
            pltpu.VMEM((_NBUF, _C // 4, _W), jnp.int32),
            pltpu.SemaphoreType.DMA((_NBUF,)),
            pltpu.SemaphoreType.DMA((_NBUF,)),
            pltpu.SemaphoreType.DMA((_NBUF,)),
        ],
    )
    return run(x, idx, m8)


def kernel(x, shuffled_idx, mask):
    return _cutmix_sc(x, shuffled_idx, mask.view(jnp.int8))


# speedup vs baseline: 1.1168x; 1.0694x over previous
import functools

import jax
import jax.numpy as jnp
from jax import lax
from jax.experimental import pallas as pl
from jax.experimental.pallas import tpu as pltpu
from jax.experimental.pallas import tpu_sc as plsc

_B = 16384
_D = 4096
_NC = 2
_NS = 16
_NW = _NC * _NS
_RPW = _B // _NW
_C = 8
_W = 2048
_NH = _D // _W
_NCH = (_RPW // _C) * _NH
_NBUF = 2
_LANES = 16


def _body(x_hbm, idx_hbm, m_hbm, out_hbm,
          idx_v, x_v, g_v, m_v, o_v, gsem, lsem, osem):
    wid = lax.axis_index("s") * _NC + lax.axis_index("c")
    base = wid * _RPW
    pltpu.sync_copy(idx_hbm.at[pl.ds(base, _RPW)], idx_v)

    m32 = m_hbm.bitcast(jnp.int32)

    def chunk_coords(ci):
        row0 = base + (ci // _NH) * _C
        col0 = pl.multiple_of((ci % _NH) * _W, _W)
        q0 = pl.multiple_of(row0 >> 2, 2)
        return row0, col0, q0

    def issue_in(ci, b):
        row0, col0, q0 = chunk_coords(ci)
        pltpu.async_copy(
            x_hbm.at[idx_v.at[pl.ds((ci // _NH) * _C, _C)],
                     pl.ds(col0, _W)],
            g_v.at[b], gsem.at[b])
        pltpu.async_copy(x_hbm.at[pl.ds(row0, _C), pl.ds(col0, _W)],
                         x_v.at[b], lsem.at[b])
        pltpu.async_copy(m32.at[pl.ds(q0, _C // 4), pl.ds(col0, _W)],
                         m_v.at[b], lsem.at[b])

    def wait_in(ci, b):
        row0, col0, q0 = chunk_coords(ci)
        pltpu.make_async_copy(
            x_hbm.at[idx_v.at[pl.ds((ci // _NH) * _C, _C)],
                     pl.ds(col0, _W)],
            g_v.at[b], gsem.at[b]).wait()
        pltpu.make_async_copy(x_hbm.at[pl.ds(row0, _C), pl.ds(col0, _W)],
                              x_v.at[b], lsem.at[b]).wait()
        pltpu.make_async_copy(m32.at[pl.ds(q0, _C // 4), pl.ds(col0, _W)],
                              m_v.at[b], lsem.at[b]).wait()

    def compute(b):
        def group(e, _):
            off = e * _LANES
            for q in range(_C // 4):
                mw = m_v[b, q, pl.ds(off, _LANES)]
                for s in range(4):
                    r = 4 * q + s
                    keep = (mw & jnp.int32(1 << (8 * s))) != 0
                    xv = x_v[b, r, pl.ds(off, _LANES)]
                    gv = g_v[b, r, pl.ds(off, _LANES)]
                    o_v[b, r, pl.ds(off, _LANES)] = jnp.where(keep, gv, xv)
            return ()

        lax.fori_loop(0, _W // _LANES, group, (), unroll=False)

    def issue_out(ci, b):
        row0, col0, _q = chunk_coords(ci)
        pltpu.async_copy(o_v.at[b],
                         out_hbm.at[pl.ds(row0, _C), pl.ds(col0, _W)],
                         osem.at[b])

    def wait_out(ci, b):
        row0, col0, _q = chunk_coords(ci)
        pltpu.make_async_copy(o_v.at[b],
                              out_hbm.at[pl.ds(row0, _C), pl.ds(col0, _W)],
                              osem.at[b]).wait()

    issue_in(0, 0)

    def step(i, _):
        for b in range(2):
            ci = 2 * i + b
            nb = 1 - b

            @pl.when(ci + 1 < _NCH)
            def _():
                issue_in(ci + 1, nb)

            wait_in(ci, b)

            @pl.when(ci >= 2)
            def _():
                wait_out(ci - 2, b)

            compute(b)
            issue_out(ci, b)
        return ()

    lax.fori_loop(0, _NCH // 2, step, (), unroll=False)
    wait_out(_NCH - 2, 0)
    wait_out(_NCH - 1, 1)


@jax.jit
def _cutmix_sc(x, idx, m8):
    mesh = plsc.VectorSubcoreMesh(core_axis_name="c", subcore_axis_name="s",
                                  num_cores=_NC, num_subcores=_NS)
    run = pl.kernel(
        _body,
        out_type=jax.ShapeDtypeStruct((_B, _D), jnp.float32),
        mesh=mesh,
        scratch_types=[
            pltpu.VMEM((_RPW,), jnp.int32),
            pltpu.VMEM((_NBUF, _C, _W), jnp.float32),
            pltpu.VMEM((_NBUF, _C, _W), jnp.float32),
            pltpu.VMEM((_NBUF, _C // 4, _W), jnp.int32),
            pltpu.VMEM((_NBUF, _C, _W), jnp.float32),
            pltpu.SemaphoreType.DMA((_NBUF,)),
            pltpu.SemaphoreType.DMA((_NBUF,)),
            pltpu.SemaphoreType.DMA((_NBUF,)),
        ],
    )
    return run(x, idx, m8)


def kernel(x, shuffled_idx, mask):
    return _cutmix_sc(x, shuffled_idx, mask.view(jnp.int8))


# speedup vs baseline: 1.1191x; 1.0020x over previous
import jax
import jax.numpy as jnp
from jax import lax
from jax.experimental import pallas as pl
from jax.experimental.pallas import tpu as pltpu
from jax.experimental.pallas import tpu_sc as plsc

_B = 16384
_D = 4096
_NC = 2
_NS = 16
_NW = _NC * _NS
_RPW = _B // _NW
_C = 8
_W = 2048
_NH = _D // _W
_NCH = (_RPW // _C) * _NH
_NBUF = 2
_LANES = 16


def _body(x_hbm, idx_hbm, m_hbm, out_hbm,
          idx_v, x_v, g_v, m_v, o_v, gsem, lsem, osem):
    wid = lax.axis_index("s") * _NC + lax.axis_index("c")
    base = wid * _RPW
    pltpu.sync_copy(idx_hbm.at[pl.ds(base, _RPW)], idx_v)

    m32 = m_hbm.bitcast(jnp.int32)

    def chunk_coords(ci):
        row0 = base + (ci // _NH) * _C
        col0 = pl.multiple_of((ci % _NH) * _W, _W)
        q0 = pl.multiple_of(row0 >> 2, 2)
        return row0, col0, q0

    def issue_in(ci, b):
        row0, col0, q0 = chunk_coords(ci)
        pltpu.async_copy(
            x_hbm.at[idx_v.at[pl.ds((ci // _NH) * _C, _C)],
                     pl.ds(col0, _W)],
            g_v.at[b], gsem.at[b])
        pltpu.async_copy(x_hbm.at[pl.ds(row0, _C), pl.ds(col0, _W)],
                         x_v.at[b], lsem.at[b])
        pltpu.async_copy(m32.at[pl.ds(q0, _C // 4), pl.ds(col0, _W)],
                         m_v.at[b], lsem.at[b])

    def wait_in(ci, b):
        row0, col0, q0 = chunk_coords(ci)
        pltpu.make_async_copy(
            x_hbm.at[idx_v.at[pl.ds((ci // _NH) * _C, _C)],
                     pl.ds(col0, _W)],
            g_v.at[b], gsem.at[b]).wait()
        pltpu.make_async_copy(x_hbm.at[pl.ds(row0, _C), pl.ds(col0, _W)],
                              x_v.at[b], lsem.at[b]).wait()
        pltpu.make_async_copy(m32.at[pl.ds(q0, _C // 4), pl.ds(col0, _W)],
                              m_v.at[b], lsem.at[b]).wait()

    def compute(b):
        def group(e, _):
            off = e * _LANES
            for q in range(_C // 4):
                mw = m_v[b, q, pl.ds(off, _LANES)]
                for s in range(4):
                    r = 4 * q + s
                    keep = (mw & jnp.int32(1 << (8 * s))) != 0
                    xv = x_v[b, r, pl.ds(off, _LANES)]
                    gv = g_v[b, r, pl.ds(off, _LANES)]
                    o_v[b, r, pl.ds(off, _LANES)] = jnp.where(keep, gv, xv)
            return ()

        lax.fori_loop(0, _W // _LANES, group, (), unroll=False)

    def issue_out(ci, b):
        row0, col0, _q = chunk_coords(ci)
        pltpu.async_copy(o_v.at[b],
                         out_hbm.at[pl.ds(row0, _C), pl.ds(col0, _W)],
                         osem.at[b])

    def wait_out(ci, b):
        row0, col0, _q = chunk_coords(ci)
        pltpu.make_async_copy(o_v.at[b],
                              out_hbm.at[pl.ds(row0, _C), pl.ds(col0, _W)],
                              osem.at[b]).wait()

    issue_in(0, 0)

    def step(i, _):
        for b in range(2):
            ci = 2 * i + b
            nb = 1 - b

            @pl.when(ci + 1 < _NCH)
            def _():
                issue_in(ci + 1, nb)

            wait_in(ci, b)

            @pl.when(ci >= 2)
            def _():
                wait_out(ci - 2, b)

            compute(b)
            issue_out(ci, b)
        return ()

    lax.fori_loop(0, _NCH // 2, step, (), unroll=False)
    wait_out(_NCH - 2, 0)
    wait_out(_NCH - 1, 1)


@jax.jit
def _cutmix_sc(x, idx, m8):
    mesh = plsc.VectorSubcoreMesh(core_axis_name="c", subcore_axis_name="s",
                                  num_cores=_NC, num_subcores=_NS)
    run = pl.kernel(
        _body,
        out_type=jax.ShapeDtypeStruct((_B, _D), jnp.float32),
        mesh=mesh,
        scratch_types=[
            pltpu.VMEM((_RPW,), jnp.int32),
            pltpu.VMEM((_NBUF, _C, _W), jnp.float32),
            pltpu.VMEM((_NBUF, _C, _W), jnp.float32),
            pltpu.VMEM((_NBUF, _C // 4, _W), jnp.int32),
            pltpu.VMEM((_NBUF, _C, _W), jnp.float32),
            pltpu.SemaphoreType.DMA((_NBUF,)),
            pltpu.SemaphoreType.DMA((_NBUF,)),
            pltpu.SemaphoreType.DMA((_NBUF,)),
        ],
    )
    return run(x, idx, m8)


def kernel(x, shuffled_idx, mask):
    return _cutmix_sc(x, shuffled_idx, mask.view(jnp.int8))
